# baseline (speedup 1.0000x reference)
import functools

import jax
import jax.numpy as jnp
from jax import lax
from jax.experimental import pallas as pl
from jax.experimental.pallas import tpu as pltpu
from jax.experimental.pallas import tpu_sc as plsc

N_NODES = 10000
N_EDGES = 320000
NFEAT = 128
NCLASS = 64
CHALF = NCLASS // 2

NC = 2
NS = 16
CHUNK = 128
EPT = 20480
NCHUNK = EPT // CHUNK
E_PAD = EPT * NS
ACC_ROWS = 10240
RPT = ACC_ROWS // NS
DUMMY_ROW = N_NODES


def _matmul_body(x_ref, w_ref, o_ref):
    h = jnp.dot(x_ref[...], w_ref[...], preferred_element_type=jnp.float32)
    o_ref[0:N_NODES, :] = h[:, 0:CHALF]
    o_ref[ACC_ROWS:ACC_ROWS + N_NODES, :] = h[:, CHALF:NCLASS]


def _matmul_split(x, w):
    return pl.pallas_call(
        _matmul_body,
        out_shape=jax.ShapeDtypeStruct((NC * ACC_ROWS, CHALF), jnp.float32),
    )(x, w)


NBUF = 12
GAHEAD = 6


def _spmm_body(hin, src_hbm, dst_hbm, out, src_v, dst_v, *rest):
    bufs = rest[:NBUF]
    acc0 = rest[NBUF]
    acc1 = rest[NBUF + 1]
    gsems = rest[NBUF + 2:NBUF + 2 + NBUF]
    ssems = rest[NBUF + 2 + NBUF:]
    rows_a = bufs[0]
    c = lax.axis_index("c")
    s = lax.axis_index("s")

    def _zero_rows(i, carry):
        rows_a[i, pl.ds(0, 16)] = jnp.zeros((16,), jnp.float32)
        rows_a[i, pl.ds(16, 16)] = jnp.zeros((16,), jnp.float32)
        return carry

    lax.fori_loop(0, CHUNK, _zero_rows, 0)

    def _zero_acc1(k, carry):
        pltpu.sync_copy(rows_a, acc1.at[pl.ds(s * RPT + k * CHUNK, CHUNK)])
        return carry

    lax.fori_loop(0, RPT // CHUNK, _zero_acc1, 0)

    pltpu.sync_copy(dst_hbm.at[pl.ds(s * NCHUNK, NCHUNK)], dst_v)
    pltpu.sync_copy(src_hbm.at[pl.ds(s * EPT, EPT)], src_v)

    pltpu.sync_copy(
        hin.at[pl.ds(c * ACC_ROWS + s * RPT, RPT)],
        acc0.at[pl.ds(s * RPT, RPT)])

    plsc.subcore_barrier()

    def _run_layer(source, acc):
        def _gather(j):
            b = (j + NBUF - GAHEAD) % NBUF
            return pltpu.async_copy(
                source.at[src_v.at[pl.ds(j * CHUNK, CHUNK)]],
                bufs[b], gsems[b])

        def _scatter(j):
            b = (j + NBUF - GAHEAD) % NBUF
            return pltpu.async_copy(
                bufs[b], acc.at[dst_v.at[j]], ssems[b], add=True)

        gath = [None] * NCHUNK
        scat = [None] * NCHUNK
        for j in range(min(GAHEAD, NCHUNK)):
            gath[j] = _gather(j)
        for j in range(NCHUNK):
            jn = j + GAHEAD
            if jn < NCHUNK:
                jp = jn - NBUF
                if jp >= 0:
                    scat[jp].wait()
                gath[jn] = _gather(jn)
            gath[j].wait()
            scat[j] = _scatter(j)
        for j in range(max(0, NCHUNK - NBUF), NCHUNK):
            scat[j].wait()

    _run_layer(acc0, acc1)
    plsc.subcore_barrier()

    def _zero_rows2(i, carry):
        rows_a[i, pl.ds(0, 16)] = jnp.zeros((16,), jnp.float32)
        rows_a[i, pl.ds(16, 16)] = jnp.zeros((16,), jnp.float32)
        return carry

    lax.fori_loop(0, CHUNK, _zero_rows2, 0)

    def _zero_acc0(k, carry):
        pltpu.sync_copy(rows_a, acc0.at[pl.ds(s * RPT + k * CHUNK, CHUNK)])
        return carry

    lax.fori_loop(0, RPT // CHUNK, _zero_acc0, 0)
    plsc.subcore_barrier()

    _run_layer(acc1, acc0)
    plsc.subcore_barrier()

    pltpu.sync_copy(
        acc0.at[pl.ds(s * RPT, RPT)],
        out.at[pl.ds(c * ACC_ROWS + s * RPT, RPT)])


_spmm = functools.partial(
    pl.kernel,
    out_type=jax.ShapeDtypeStruct((NC * ACC_ROWS, CHALF), jnp.float32),
    mesh=plsc.VectorSubcoreMesh(core_axis_name="c", subcore_axis_name="s"),
    scratch_types=[
        pltpu.VMEM((EPT,), jnp.int32),
        pltpu.VMEM((NCHUNK, CHUNK), jnp.int32),
        *[pltpu.VMEM((CHUNK, CHALF), jnp.float32) for _ in range(NBUF)],
        pltpu.VMEM_SHARED((ACC_ROWS, CHALF), jnp.float32),
        pltpu.VMEM_SHARED((ACC_ROWS, CHALF), jnp.float32),
        *[pltpu.SemaphoreType.DMA for _ in range(2 * NBUF)],
    ],
    compiler_params=pltpu.CompilerParams(use_tc_tiling_on_sc=False),
)(_spmm_body)


def _finish_body(p_ref, b_ref, o_ref):
    h = jnp.concatenate(
        [p_ref[0:N_NODES, :], p_ref[ACC_ROWS:ACC_ROWS + N_NODES, :]], axis=1)
    h = h + b_ref[...]
    m = jnp.max(h, axis=1, keepdims=True)
    e = jnp.exp(h - m)
    lse = jnp.log(jnp.sum(e, axis=1, keepdims=True))
    o_ref[...] = h - m - lse


def _finish(p, b2):
    return pl.pallas_call(
        _finish_body,
        out_shape=jax.ShapeDtypeStruct((N_NODES, NCLASS), jnp.float32),
    )(p, b2)


def kernel(x, edge_index, W, b):
    src = edge_index[0]
    dst = edge_index[1]
    pad = E_PAD - N_EDGES
    src_pad = jnp.concatenate([src, jnp.zeros((pad,), jnp.int32)])
    dst_pad = jnp.concatenate(
        [dst, jnp.full((pad,), DUMMY_ROW, jnp.int32)])
    dst2d = dst_pad.reshape(E_PAD // CHUNK, CHUNK)

    h = _matmul_split(x, W)
    h = _spmm(h, src_pad, dst2d)
    return _finish(h, b.reshape(1, NCLASS))


# speedup vs baseline: 1.0251x; 1.0251x over previous
import functools

import jax
import jax.numpy as jnp
from jax import lax
from jax.experimental import pallas as pl
from jax.experimental.pallas import tpu as pltpu
from jax.experimental.pallas import tpu_sc as plsc

N_NODES = 10000
N_EDGES = 320000
NFEAT = 128
NCLASS = 64
CHALF = NCLASS // 2

NC = 2
NS = 16
CHUNK = 128
EPT = 20480
NCHUNK = EPT // CHUNK
E_PAD = EPT * NS
ACC_ROWS = 10240
RPT = ACC_ROWS // NS
DUMMY_ROW = N_NODES


def _matmul_body(x_ref, w_ref, o_ref):
    h = jnp.dot(x_ref[...], w_ref[...], preferred_element_type=jnp.float32)
    o_ref[0:N_NODES, :] = h[:, 0:CHALF]
    o_ref[ACC_ROWS:ACC_ROWS + N_NODES, :] = h[:, CHALF:NCLASS]


def _matmul_split(x, w):
    return pl.pallas_call(
        _matmul_body,
        out_shape=jax.ShapeDtypeStruct((NC * ACC_ROWS, CHALF), jnp.float32),
    )(x, w)


NBUF = 8
GAHEAD = 4


def _spmm_body(hin, src_hbm, dst_hbm, out, src_v, dst_v, zbuf, *rest):
    bufs = rest[:NBUF]
    acc0 = rest[NBUF]
    acc1 = rest[NBUF + 1]
    gsems = rest[NBUF + 2:NBUF + 2 + NBUF]
    ssems = rest[NBUF + 2 + NBUF:]
    c = lax.axis_index("c")
    s = lax.axis_index("s")

    def _zero_rows(i, carry):
        zbuf[i, pl.ds(0, 16)] = jnp.zeros((16,), jnp.float32)
        zbuf[i, pl.ds(16, 16)] = jnp.zeros((16,), jnp.float32)
        return carry

    lax.fori_loop(0, CHUNK, _zero_rows, 0)

    stage = [
        pltpu.async_copy(
            dst_hbm.at[pl.ds(s * NCHUNK, NCHUNK)], dst_v, gsems[0]),
        pltpu.async_copy(
            src_hbm.at[pl.ds(s * EPT, EPT)], src_v, gsems[1]),
        pltpu.async_copy(
            hin.at[pl.ds(c * ACC_ROWS + s * RPT, RPT)],
            acc0.at[pl.ds(s * RPT, RPT)], gsems[2]),
    ]
    stage += [
        pltpu.async_copy(
            zbuf, acc1.at[pl.ds(s * RPT + k * CHUNK, CHUNK)], ssems[k])
        for k in range(RPT // CHUNK)
    ]
    for d in stage:
        d.wait()

    plsc.subcore_barrier()

    def _run_layer(source, acc):
        def _gather(j):
            b = (j + NBUF - GAHEAD) % NBUF
            return pltpu.async_copy(
                source.at[src_v.at[pl.ds(j * CHUNK, CHUNK)]],
                bufs[b], gsems[b])

        def _scatter(j):
            b = (j + NBUF - GAHEAD) % NBUF
            return pltpu.async_copy(
                bufs[b], acc.at[dst_v.at[j]], ssems[b], add=True)

        gath = [None] * NCHUNK
        scat = [None] * NCHUNK
        for j in range(min(GAHEAD, NCHUNK)):
            gath[j] = _gather(j)
        for j in range(NCHUNK):
            jn = j + GAHEAD
            if jn < NCHUNK:
                jp = jn - NBUF
                if jp >= 0:
                    scat[jp].wait()
                gath[jn] = _gather(jn)
            gath[j].wait()
            scat[j] = _scatter(j)
        for j in range(max(0, NCHUNK - NBUF), NCHUNK):
            scat[j].wait()

    _run_layer(acc0, acc1)
    plsc.subcore_barrier()

    zero2 = [
        pltpu.async_copy(
            zbuf, acc0.at[pl.ds(s * RPT + k * CHUNK, CHUNK)], ssems[k])
        for k in range(RPT // CHUNK)
    ]
    for d in zero2:
        d.wait()
    plsc.subcore_barrier()

    _run_layer(acc1, acc0)
    plsc.subcore_barrier()

    pltpu.sync_copy(
        acc0.at[pl.ds(s * RPT, RPT)],
        out.at[pl.ds(c * ACC_ROWS + s * RPT, RPT)])


_spmm = functools.partial(
    pl.kernel,
    out_type=jax.ShapeDtypeStruct((NC * ACC_ROWS, CHALF), jnp.float32),
    mesh=plsc.VectorSubcoreMesh(core_axis_name="c", subcore_axis_name="s"),
    scratch_types=[
        pltpu.VMEM((EPT,), jnp.int32),
        pltpu.VMEM((NCHUNK, CHUNK), jnp.int32),
        pltpu.VMEM((CHUNK, CHALF), jnp.float32),
        *[pltpu.VMEM((CHUNK, CHALF), jnp.float32) for _ in range(NBUF)],
        pltpu.VMEM_SHARED((ACC_ROWS, CHALF), jnp.float32),
        pltpu.VMEM_SHARED((ACC_ROWS, CHALF), jnp.float32),
        *[pltpu.SemaphoreType.DMA for _ in range(2 * NBUF)],
    ],
    compiler_params=pltpu.CompilerParams(use_tc_tiling_on_sc=False),
)(_spmm_body)


def _finish_body(p_ref, b_ref, o_ref):
    h = jnp.concatenate(
        [p_ref[0:N_NODES, :], p_ref[ACC_ROWS:ACC_ROWS + N_NODES, :]], axis=1)
    h = h + b_ref[...]
    m = jnp.max(h, axis=1, keepdims=True)
    e = jnp.exp(h - m)
    lse = jnp.log(jnp.sum(e, axis=1, keepdims=True))
    o_ref[...] = h - m - lse


def _finish(p, b2):
    return pl.pallas_call(
        _finish_body,
        out_shape=jax.ShapeDtypeStruct((N_NODES, NCLASS), jnp.float32),
    )(p, b2)


def kernel(x, edge_index, W, b):
    src = edge_index[0]
    dst = edge_index[1]
    pad = E_PAD - N_EDGES
    src_pad = jnp.concatenate([src, jnp.zeros((pad,), jnp.int32)])
    dst_pad = jnp.concatenate(
        [dst, jnp.full((pad,), DUMMY_ROW, jnp.int32)])
    dst2d = dst_pad.reshape(E_PAD // CHUNK, CHUNK)

    h = _matmul_split(x, W)
    h = _spmm(h, src_pad, dst2d)
    return _finish(h, b.reshape(1, NCLASS))
